# baseline (speedup 1.0000x reference)
import numpy as np
import jax
import jax.numpy as jnp
from jax import lax
from jax.experimental import pallas as pl
from jax.experimental.pallas import tpu as pltpu
from jax.experimental.pallas import tpu_sc as plsc

B, S, Q, D = 8, 64, 100, 128
E = 512000
N = B * S * Q
QP = 128
ND = B * S
DEG_W = ND * QP
QTR = N // 4
QTR_W = QTR * QP
A_W = N * QP

NC, NS = 2, 16
CHUNK = 128
EPT_DEG = E // (NC * NS)
EPT_A = E // NS
ZB = 4096

_mesh = plsc.VectorSubcoreMesh(core_axis_name="c", subcore_axis_name="s")


def _fill(ref, n, vec):
    def body(j, _):
        ref[pl.ds(j * 16, 16)] = vec
        return 0
    lax.fori_loop(0, n // 16, body, 0)


DEG_ROWS = EPT_DEG // CHUNK


def _deg_body(edge_hbm, out_hbm, deg_s, cbuf, idxb, ones_v, zbuf, tmp, sem):
    cid = lax.axis_index("c")
    sid = lax.axis_index("s")
    _fill(zbuf, ZB, jnp.zeros((16,), jnp.float32))
    _fill(ones_v, CHUNK, jnp.ones((16,), jnp.float32))
    pltpu.sync_copy(zbuf, deg_s.at[pl.ds(sid * ZB, ZB)])
    plsc.subcore_barrier()
    base = cid * (E // NC) + sid * EPT_DEG
    pltpu.sync_copy(edge_hbm.at[pl.ds(E + base, EPT_DEG)], cbuf)

    def row(r, _):
        for j in range(CHUNK // 16):
            c = cbuf[pl.ds(r * CHUNK + j * 16, 16)]
            didx = lax.div(c, Q) * QP + lax.rem(c, Q)
            idxb[r, pl.ds(j * 16, 16)] = didx
        pltpu.async_copy(ones_v, deg_s.at[idxb.at[r]], sem, add=True)
        return 0

    lax.fori_loop(0, DEG_ROWS, row, 0)

    def drain(r, _):
        pltpu.make_async_copy(ones_v, deg_s.at[idxb.at[0]], sem).wait()
        return 0

    lax.fori_loop(0, DEG_ROWS, drain, 0)
    plsc.subcore_barrier()
    pltpu.sync_copy(deg_s.at[pl.ds(sid * ZB, ZB)], tmp)
    pltpu.sync_copy(tmp, out_hbm.at[cid, pl.ds(sid * ZB, ZB)])


_deg_call = pl.kernel(
    _deg_body,
    out_type=jax.ShapeDtypeStruct((NC, DEG_W), jnp.float32),
    mesh=_mesh,
    compiler_params=pltpu.CompilerParams(needs_layout_passes=False),
    scratch_types=[
        pltpu.VMEM_SHARED((DEG_W,), jnp.float32),
        pltpu.VMEM((EPT_DEG,), jnp.int32),
        pltpu.VMEM((DEG_ROWS, CHUNK), jnp.int32),
        pltpu.VMEM((CHUNK,), jnp.float32),
        pltpu.VMEM((ZB,), jnp.float32),
        pltpu.VMEM((ZB,), jnp.float32),
        pltpu.SemaphoreType.DMA,
    ],
)


EPW = E // (NC * NS)
PREP_C = 4000


def _edges_body(edge_hbm, dis_hbm, p_hbm, w_hbm,
                dis_v, rbuf, cbuf, pbuf, wbuf, isem, osem):
    cid = lax.axis_index("c")
    sid = lax.axis_index("s")
    wid = cid * NS + sid
    nparts = EPW // PREP_C

    def fire_in(part):
        base = wid * EPW + part * PREP_C
        bi = part % 2
        pltpu.async_copy(edge_hbm.at[pl.ds(base, PREP_C)],
                         rbuf.at[pl.ds(bi * PREP_C, PREP_C)], isem)
        pltpu.async_copy(edge_hbm.at[pl.ds(E + base, PREP_C)],
                         cbuf.at[pl.ds(bi * PREP_C, PREP_C)], isem)

    fire_in(0)
    pltpu.sync_copy(dis_hbm, dis_v)
    for part in range(nparts):
        base = wid * EPW + part * PREP_C
        bi = part % 2
        pltpu.make_async_copy(edge_hbm.at[pl.ds(base, PREP_C)],
                              rbuf.at[pl.ds(bi * PREP_C, PREP_C)], isem).wait()
        pltpu.make_async_copy(edge_hbm.at[pl.ds(E + base, PREP_C)],
                              cbuf.at[pl.ds(bi * PREP_C, PREP_C)], isem).wait()
        if part + 1 < nparts:
            fire_in(part + 1)
        if part >= 2:
            pltpu.make_async_copy(
                pbuf.at[pl.ds(bi * PREP_C, PREP_C)],
                p_hbm.at[pl.ds(0, PREP_C)], osem).wait()
            pltpu.make_async_copy(
                wbuf.at[pl.ds(bi * PREP_C, PREP_C)],
                w_hbm.at[pl.ds(0, PREP_C)], osem).wait()

        def step(k, _):
            for j in range(5):
                sl = pl.ds(bi * PREP_C + k * 80 + j * 16, 16)
                r = rbuf[sl]
                c = cbuf[sl]
                q = lax.rem(r, Q)
                ridx = lax.div(r, Q) * QP + q
                cidx = lax.div(c, Q) * QP + lax.rem(c, Q)
                w = (plsc.load_gather(dis_v, [ridx])
                     * plsc.load_gather(dis_v, [cidx]))
                pbuf[sl] = c * QP + q
                wbuf[sl] = w
            return 0

        lax.fori_loop(0, PREP_C // 80, step, 0)
        pltpu.async_copy(pbuf.at[pl.ds(bi * PREP_C, PREP_C)],
                         p_hbm.at[pl.ds(base, PREP_C)], osem)
        pltpu.async_copy(wbuf.at[pl.ds(bi * PREP_C, PREP_C)],
                         w_hbm.at[pl.ds(base, PREP_C)], osem)
    for _ in range(2 * min(2, nparts)):
        pltpu.make_async_copy(
            pbuf.at[pl.ds(0, PREP_C)], p_hbm.at[pl.ds(0, PREP_C)], osem).wait()


_edges_call = pl.kernel(
    _edges_body,
    out_type=(jax.ShapeDtypeStruct((E,), jnp.int32),
              jax.ShapeDtypeStruct((E,), jnp.float32)),
    mesh=_mesh,
    compiler_params=pltpu.CompilerParams(needs_layout_passes=False),
    scratch_types=[
        pltpu.VMEM((DEG_W,), jnp.float32),
        pltpu.VMEM((2 * PREP_C,), jnp.int32),
        pltpu.VMEM((2 * PREP_C,), jnp.int32),
        pltpu.VMEM((2 * PREP_C,), jnp.int32),
        pltpu.VMEM((2 * PREP_C,), jnp.float32),
        pltpu.SemaphoreType.DMA,
        pltpu.SemaphoreType.DMA,
    ],
)

SC_C = 3200
SCHUNK = 80
NROW = SC_C // SCHUNK
ZB2 = 2048


def _scatter_body(p_hbm, w_hbm, a_hbm, a_s, pbuf, wbuf, idxb, wb, zbuf,
                  sem, zsem, osem, isem):
    cid = lax.axis_index("c")
    sid = lax.axis_index("s")
    _fill(zbuf, ZB2, jnp.zeros((16,), jnp.float32))
    spw = QTR_W // NS

    for quarter in range(2):
        qid = cid * 2 + quarter

        def zinit(k, _):
            pltpu.async_copy(zbuf, a_s.at[pl.ds(sid * spw + k * ZB2, ZB2)],
                             zsem)
            return 0

        lax.fori_loop(0, spw // ZB2, zinit, 0)

        def zdrain(k, _):
            pltpu.make_async_copy(
                zbuf, a_s.at[pl.ds(sid * spw, ZB2)], zsem).wait()
            return 0

        lax.fori_loop(0, spw // ZB2, zdrain, 0)
        plsc.subcore_barrier()

        nparts = EPT_A // SC_C

        def fire_in(part):
            base = sid * EPT_A + part * SC_C
            bi = part % 2
            pltpu.async_copy(p_hbm.at[pl.ds(base, SC_C)],
                             pbuf.at[pl.ds(bi * SC_C, SC_C)], isem)
            pltpu.async_copy(w_hbm.at[pl.ds(base, SC_C)],
                             wbuf.at[pl.ds(bi * SC_C, SC_C)], isem)

        fire_in(0)
        for part in range(nparts):
            base = sid * EPT_A + part * SC_C
            bi = part % 2
            pltpu.make_async_copy(p_hbm.at[pl.ds(base, SC_C)],
                                  pbuf.at[pl.ds(bi * SC_C, SC_C)], isem).wait()
            pltpu.make_async_copy(w_hbm.at[pl.ds(base, SC_C)],
                                  wbuf.at[pl.ds(bi * SC_C, SC_C)], isem).wait()
            if part + 1 < nparts:
                fire_in(part + 1)

            def row(r, _):
                for j in range(SCHUNK // 16):
                    sl = pl.ds(bi * SC_C + r * SCHUNK + j * 16, 16)
                    p = pbuf[sl]
                    w = wbuf[sl]
                    w = jnp.where(lax.div(p, QTR_W) == qid, w, 0.0)
                    idxb[r, pl.ds(j * 16, 16)] = lax.rem(p, QTR_W)
                    wb[r, pl.ds(j * 16, 16)] = w
                pltpu.async_copy(wb.at[r], a_s.at[idxb.at[r]], sem, add=True)
                return 0

            lax.fori_loop(0, NROW, row, 0)

            def drain(r, _):
                pltpu.make_async_copy(wb.at[0], a_s.at[idxb.at[0]],
                                      sem).wait()
                return 0

            lax.fori_loop(0, NROW, drain, 0)

        plsc.subcore_barrier()
        pltpu.async_copy(a_s.at[pl.ds(sid * spw, spw)],
                         a_hbm.at[pl.ds(qid * QTR_W + sid * spw, spw)], osem)
        pltpu.make_async_copy(
            a_s.at[pl.ds(sid * spw, spw)],
            a_hbm.at[pl.ds(qid * QTR_W + sid * spw, spw)], osem).wait()
        plsc.subcore_barrier()


_a_call = pl.kernel(
    _scatter_body,
    out_type=jax.ShapeDtypeStruct((A_W,), jnp.float32),
    mesh=_mesh,
    compiler_params=pltpu.CompilerParams(needs_layout_passes=False),
    scratch_types=[
        pltpu.VMEM_SHARED((QTR_W,), jnp.float32),
        pltpu.VMEM((2 * SC_C,), jnp.int32),
        pltpu.VMEM((2 * SC_C,), jnp.float32),
        pltpu.VMEM((NROW, SCHUNK), jnp.int32),
        pltpu.VMEM((NROW, SCHUNK), jnp.float32),
        pltpu.VMEM((ZB2,), jnp.float32),
        pltpu.SemaphoreType.DMA,
        pltpu.SemaphoreType.DMA,
        pltpu.SemaphoreType.DMA,
        pltpu.SemaphoreType.DMA,
    ],
)


def _prep_body(dp_ref, emb_ref, w_ref, dis_ref, xq_ref):
    deg = dp_ref[0] + dp_ref[1]
    dis_ref[...] = jnp.where(
        deg > 0.0, lax.rsqrt(jnp.where(deg > 0.0, deg, 1.0)), 0.0)
    xq_ref[...] = jnp.dot(emb_ref[...], w_ref[...],
                          preferred_element_type=jnp.float32)


def _mm_body(a_ref, xq_ref, b_ref, pe_ref, out_ref):
    out_ref[0] = (jnp.dot(a_ref[0], xq_ref[...],
                          preferred_element_type=jnp.float32)
                  + b_ref[...] + pe_ref[0])


def _pos_encoding():
    pos = np.arange(S, dtype=np.float32)[:, None]
    div = np.exp(np.arange(0, D, 2, dtype=np.float32) * (-np.log(10000.0) / D))
    pe = np.zeros((S, D), dtype=np.float32)
    pe[:, 0::2] = np.sin(pos * div)
    pe[:, 1::2] = np.cos(pos * div)
    return jnp.asarray(pe)


def kernel(edge_index, emb_table, W, b):
    emb_pad = jnp.zeros((QP, D), jnp.float32).at[:Q].set(emb_table[:Q])

    ei_flat = edge_index.reshape(-1)
    deg_part = _deg_call(ei_flat)

    dis, xq = pl.pallas_call(
        _prep_body,
        out_shape=(
            jax.ShapeDtypeStruct((ND, QP), jnp.float32),
            jax.ShapeDtypeStruct((QP, D), jnp.float32),
        ),
    )(deg_part.reshape(NC, ND, QP), emb_pad, W)

    packed, wvals = _edges_call(ei_flat, dis.reshape(-1))
    a_flat = _a_call(packed, wvals)

    out = pl.pallas_call(
        _mm_body,
        grid=(ND,),
        in_specs=[
            pl.BlockSpec((1, Q, QP), lambda i: (i, 0, 0)),
            pl.BlockSpec((QP, D), lambda i: (0, 0)),
            pl.BlockSpec((1, D), lambda i: (0, 0)),
            pl.BlockSpec((1, 1, D), lambda i: (i % S, 0, 0)),
        ],
        out_specs=pl.BlockSpec((1, Q, D), lambda i: (i, 0, 0)),
        out_shape=jax.ShapeDtypeStruct((ND, Q, D), jnp.float32),
    )(a_flat.reshape(ND, Q, QP), xq, b.reshape(1, D),
      _pos_encoding().reshape(S, 1, D))

    return out.reshape(B, S, Q, D)


# speedup vs baseline: 1.1929x; 1.1929x over previous
import numpy as np
import jax
import jax.numpy as jnp
from jax import lax
from jax.experimental import pallas as pl
from jax.experimental.pallas import tpu as pltpu
from jax.experimental.pallas import tpu_sc as plsc

B, S, Q, D = 8, 64, 100, 128
E = 512000
N = B * S * Q
QP = 128
ND = B * S
DEG_W = ND * QP
QTR = N // 4
QTR_W = QTR * QP
A_W = N * QP

NC, NS = 2, 16
CHUNK = 128
EPT_DEG = E // (NC * NS)
EPT_A = E // NS
ZB = 4096

_mesh = plsc.VectorSubcoreMesh(core_axis_name="c", subcore_axis_name="s")


def _fill(ref, n, vec):
    def body(j, _):
        ref[pl.ds(j * 16, 16)] = vec
        return 0
    lax.fori_loop(0, n // 16, body, 0)


DEG_ROWS = EPT_DEG // CHUNK


def _deg_body(edge_hbm, out_hbm, deg_s, cbuf, idxb, ones_v, zbuf, tmp, sem):
    cid = lax.axis_index("c")
    sid = lax.axis_index("s")
    _fill(zbuf, ZB, jnp.zeros((16,), jnp.float32))
    _fill(ones_v, CHUNK, jnp.ones((16,), jnp.float32))
    pltpu.sync_copy(zbuf, deg_s.at[pl.ds(sid * ZB, ZB)])
    plsc.subcore_barrier()
    base = cid * (E // NC) + sid * EPT_DEG
    pltpu.sync_copy(edge_hbm.at[pl.ds(E + base, EPT_DEG)], cbuf)

    def row(r, _):
        for j in range(CHUNK // 16):
            c = cbuf[pl.ds(r * CHUNK + j * 16, 16)]
            didx = lax.div(c, Q) * QP + lax.rem(c, Q)
            idxb[r, pl.ds(j * 16, 16)] = didx
        pltpu.async_copy(ones_v, deg_s.at[idxb.at[r]], sem, add=True)
        return 0

    lax.fori_loop(0, DEG_ROWS, row, 0)

    def drain(r, _):
        pltpu.make_async_copy(ones_v, deg_s.at[idxb.at[0]], sem).wait()
        return 0

    lax.fori_loop(0, DEG_ROWS, drain, 0)
    plsc.subcore_barrier()
    pltpu.sync_copy(deg_s.at[pl.ds(sid * ZB, ZB)], tmp)
    pltpu.sync_copy(tmp, out_hbm.at[cid, pl.ds(sid * ZB, ZB)])


_deg_call = pl.kernel(
    _deg_body,
    out_type=jax.ShapeDtypeStruct((NC, DEG_W), jnp.float32),
    mesh=_mesh,
    compiler_params=pltpu.CompilerParams(needs_layout_passes=False),
    scratch_types=[
        pltpu.VMEM_SHARED((DEG_W,), jnp.float32),
        pltpu.VMEM((EPT_DEG,), jnp.int32),
        pltpu.VMEM((DEG_ROWS, CHUNK), jnp.int32),
        pltpu.VMEM((CHUNK,), jnp.float32),
        pltpu.VMEM((ZB,), jnp.float32),
        pltpu.VMEM((ZB,), jnp.float32),
        pltpu.SemaphoreType.DMA,
    ],
)


EPW = E // (NC * NS)
PREP_C = 4000
NSP = EPW // PREP_C
SEG = PREP_C + 80
NSEG = NC * NS * NSP
CNT_S = 8


def _edges_body(edge_hbm, dis_hbm, p_hbm, w_hbm, cnt_hbm,
                dis_v, rbuf, cbuf, pb4, wb4, cv, osem):
    cid = lax.axis_index("c")
    sid = lax.axis_index("s")
    wid = cid * NS + sid
    pltpu.sync_copy(dis_hbm, dis_v)
    z16i = jnp.zeros((16,), jnp.int32)
    z16f = jnp.zeros((16,), jnp.float32)

    for sp in range(NSP):
        base = wid * EPW + sp * PREP_C
        pltpu.sync_copy(edge_hbm.at[pl.ds(base, PREP_C)], rbuf)
        pltpu.sync_copy(edge_hbm.at[pl.ds(E + base, PREP_C)], cbuf)

        def step(k, offs):
            sl = pl.ds(k * 16, 16)
            r = rbuf[sl]
            c = cbuf[sl]
            q = lax.rem(r, Q)
            ridx = lax.div(r, Q) * QP + q
            cidx = lax.div(c, Q) * QP + lax.rem(c, Q)
            w = (plsc.load_gather(dis_v, [ridx])
                 * plsc.load_gather(dis_v, [cidx]))
            p = c * QP + q
            qv = lax.div(p, QTR_W)
            new = []
            for qq in range(4):
                m = qv == qq
                plsc.store_compressed(pb4.at[pl.ds(qq * SEG + offs[qq], 16)],
                                      p, mask=m)
                plsc.store_compressed(wb4.at[pl.ds(qq * SEG + offs[qq], 16)],
                                      w, mask=m)
                new.append(offs[qq]
                           + jnp.max(plsc.all_reduce_population_count(m)))
            return tuple(new)

        offs = lax.fori_loop(0, PREP_C // 16, step, (0, 0, 0, 0))

        seg_id = (wid * NSP + sp) * 4
        for qq in range(4):
            for j in range(5):
                pb4[pl.ds(qq * SEG + offs[qq] + j * 16, 16)] = z16i
                wb4[pl.ds(qq * SEG + offs[qq] + j * 16, 16)] = z16f
            dst = (seg_id + qq) * SEG
            pltpu.async_copy(pb4.at[pl.ds(qq * SEG, SEG)],
                             p_hbm.at[pl.ds(dst, SEG)], osem)
            pltpu.async_copy(wb4.at[pl.ds(qq * SEG, SEG)],
                             w_hbm.at[pl.ds(dst, SEG)], osem)
            cv[...] = z16i + offs[qq]
            pltpu.sync_copy(cv.at[pl.ds(0, CNT_S)],
                            cnt_hbm.at[pl.ds((seg_id + qq) * CNT_S, CNT_S)])
        for qq in range(4):
            pltpu.make_async_copy(pb4.at[pl.ds(qq * SEG, SEG)],
                                  p_hbm.at[pl.ds(0, SEG)], osem).wait()
            pltpu.make_async_copy(wb4.at[pl.ds(qq * SEG, SEG)],
                                  w_hbm.at[pl.ds(0, SEG)], osem).wait()


_edges_call = pl.kernel(
    _edges_body,
    out_type=(jax.ShapeDtypeStruct((NSEG * 4 * SEG,), jnp.int32),
              jax.ShapeDtypeStruct((NSEG * 4 * SEG,), jnp.float32),
              jax.ShapeDtypeStruct((NSEG * 4 * CNT_S,), jnp.int32)),
    mesh=_mesh,
    compiler_params=pltpu.CompilerParams(needs_layout_passes=False),
    scratch_types=[
        pltpu.VMEM((DEG_W,), jnp.float32),
        pltpu.VMEM((PREP_C,), jnp.int32),
        pltpu.VMEM((PREP_C,), jnp.int32),
        pltpu.VMEM((4 * SEG,), jnp.int32),
        pltpu.VMEM((4 * SEG,), jnp.float32),
        pltpu.VMEM((16,), jnp.int32),
        pltpu.SemaphoreType.DMA,
    ],
)

SCHUNK = 80
SRING = 16
ZB2 = 2048
SEGT = NSEG // NS


def _scatter_body(p_hbm, w_hbm, cnt_hbm, a_hbm, a_s, pbuf, wbuf, idxb, wb,
                  zbuf, cntv, sem, zsem, osem, isem):
    cid = lax.axis_index("c")
    sid = lax.axis_index("s")
    _fill(zbuf, ZB2, jnp.zeros((16,), jnp.float32))
    pltpu.sync_copy(cnt_hbm, cntv.at[pl.ds(0, NSEG * 4 * CNT_S)])
    spw = QTR_W // NS

    for quarter in range(2):
        qid = cid * 2 + quarter

        def zinit(k, _):
            pltpu.async_copy(zbuf, a_s.at[pl.ds(sid * spw + k * ZB2, ZB2)],
                             zsem)
            return 0

        lax.fori_loop(0, spw // ZB2, zinit, 0)

        def zdrain(k, _):
            pltpu.make_async_copy(
                zbuf, a_s.at[pl.ds(sid * spw, ZB2)], zsem).wait()
            return 0

        lax.fori_loop(0, spw // ZB2, zdrain, 0)
        plsc.subcore_barrier()

        def seg_src(i):
            seg_id = sid * SEGT + i
            return (seg_id * 4 + qid) * SEG

        def fire_in(i):
            bi = i % 2
            pltpu.async_copy(p_hbm.at[pl.ds(seg_src(i), SEG)],
                             pbuf.at[pl.ds(bi * SEG, SEG)], isem)
            pltpu.async_copy(w_hbm.at[pl.ds(seg_src(i), SEG)],
                             wbuf.at[pl.ds(bi * SEG, SEG)], isem)

        fire_in(0)
        for i in range(SEGT):
            bi = i % 2
            pltpu.make_async_copy(p_hbm.at[pl.ds(seg_src(i), SEG)],
                                  pbuf.at[pl.ds(bi * SEG, SEG)], isem).wait()
            pltpu.make_async_copy(w_hbm.at[pl.ds(seg_src(i), SEG)],
                                  wbuf.at[pl.ds(bi * SEG, SEG)], isem).wait()
            if i + 1 < SEGT:
                fire_in(i + 1)
            seg_id = sid * SEGT + i
            cvec = cntv[pl.ds((seg_id * 4 + qid) * CNT_S, 16)]
            c = cvec[0]
            nch = lax.div(c + SCHUNK - 1, SCHUNK)

            def row(r, _):
                br = lax.rem(r, SRING)

                @pl.when(r >= SRING)
                def _():
                    pltpu.make_async_copy(wb.at[0], a_s.at[idxb.at[0]],
                                          sem).wait()

                for j in range(SCHUNK // 16):
                    sl = pl.ds(bi * SEG + r * SCHUNK + j * 16, 16)
                    p = pbuf[sl]
                    idxb[br, pl.ds(j * 16, 16)] = lax.rem(p, QTR_W)
                    wb[br, pl.ds(j * 16, 16)] = wbuf[sl]
                pltpu.async_copy(wb.at[br], a_s.at[idxb.at[br]], sem,
                                 add=True)
                return 0

            lax.fori_loop(0, nch, row, 0)

            def drain(r, _):
                @pl.when(r < nch)
                def _():
                    pltpu.make_async_copy(wb.at[0], a_s.at[idxb.at[0]],
                                          sem).wait()
                return 0

            lax.fori_loop(0, SRING, drain, 0)

        plsc.subcore_barrier()
        pltpu.async_copy(a_s.at[pl.ds(sid * spw, spw)],
                         a_hbm.at[pl.ds(qid * QTR_W + sid * spw, spw)], osem)
        pltpu.make_async_copy(
            a_s.at[pl.ds(sid * spw, spw)],
            a_hbm.at[pl.ds(qid * QTR_W + sid * spw, spw)], osem).wait()
        plsc.subcore_barrier()


_a_call = pl.kernel(
    _scatter_body,
    out_type=jax.ShapeDtypeStruct((A_W,), jnp.float32),
    mesh=_mesh,
    compiler_params=pltpu.CompilerParams(needs_layout_passes=False),
    scratch_types=[
        pltpu.VMEM_SHARED((QTR_W,), jnp.float32),
        pltpu.VMEM((2 * SEG,), jnp.int32),
        pltpu.VMEM((2 * SEG,), jnp.float32),
        pltpu.VMEM((SRING, SCHUNK), jnp.int32),
        pltpu.VMEM((SRING, SCHUNK), jnp.float32),
        pltpu.VMEM((ZB2,), jnp.float32),
        pltpu.VMEM((NSEG * 4 * CNT_S + 16,), jnp.int32),
        pltpu.SemaphoreType.DMA,
        pltpu.SemaphoreType.DMA,
        pltpu.SemaphoreType.DMA,
        pltpu.SemaphoreType.DMA,
    ],
)


def _prep_body(dp_ref, emb_ref, w_ref, dis_ref, xq_ref):
    deg = dp_ref[0] + dp_ref[1]
    dis_ref[...] = jnp.where(
        deg > 0.0, lax.rsqrt(jnp.where(deg > 0.0, deg, 1.0)), 0.0)
    xq_ref[...] = jnp.dot(emb_ref[...], w_ref[...],
                          preferred_element_type=jnp.float32)


def _mm_body(a_ref, xq_ref, b_ref, pe_ref, out_ref):
    out_ref[0] = (jnp.dot(a_ref[0], xq_ref[...],
                          preferred_element_type=jnp.float32)
                  + b_ref[...] + pe_ref[0])


def _pos_encoding():
    pos = np.arange(S, dtype=np.float32)[:, None]
    div = np.exp(np.arange(0, D, 2, dtype=np.float32) * (-np.log(10000.0) / D))
    pe = np.zeros((S, D), dtype=np.float32)
    pe[:, 0::2] = np.sin(pos * div)
    pe[:, 1::2] = np.cos(pos * div)
    return jnp.asarray(pe)


def kernel(edge_index, emb_table, W, b):
    emb_pad = jnp.zeros((QP, D), jnp.float32).at[:Q].set(emb_table[:Q])

    ei_flat = edge_index.reshape(-1)
    deg_part = _deg_call(ei_flat)

    dis, xq = pl.pallas_call(
        _prep_body,
        out_shape=(
            jax.ShapeDtypeStruct((ND, QP), jnp.float32),
            jax.ShapeDtypeStruct((QP, D), jnp.float32),
        ),
    )(deg_part.reshape(NC, ND, QP), emb_pad, W)

    packed, wvals, cnts = _edges_call(ei_flat, dis.reshape(-1))
    a_flat = _a_call(packed, wvals, cnts)

    out = pl.pallas_call(
        _mm_body,
        grid=(ND,),
        in_specs=[
            pl.BlockSpec((1, Q, QP), lambda i: (i, 0, 0)),
            pl.BlockSpec((QP, D), lambda i: (0, 0)),
            pl.BlockSpec((1, D), lambda i: (0, 0)),
            pl.BlockSpec((1, 1, D), lambda i: (i % S, 0, 0)),
        ],
        out_specs=pl.BlockSpec((1, Q, D), lambda i: (i, 0, 0)),
        out_shape=jax.ShapeDtypeStruct((ND, Q, D), jnp.float32),
    )(a_flat.reshape(ND, Q, QP), xq, b.reshape(1, D),
      _pos_encoding().reshape(S, 1, D))

    return out.reshape(B, S, Q, D)
